# baseline (device time: 886317 ns/iter reference)
import jax
import jax.numpy as jnp
from jax import lax
from jax.experimental import pallas as pl
from jax.experimental.pallas import tpu as pltpu

N_DEV = 16


def kernel(x, w_mat):
    M, K = x.shape
    _, N = w_mat.shape
    CH = M // N_DEV
    N2 = N // 2
    N_STEPS = 2 * (N_DEV - 1)
    NSLOT = 4

    def body(x_ref, w_ref, out_ref,
             comm_a, comm_b, stage_a, stage_b,
             send_a, recv_a, send_b, recv_b,
             credit_a, credit_b, osem_a, osem_b):
        me = lax.axis_index("i")
        left = (me - 1) % N_DEV
        right = (me + 1) % N_DEV

        barrier = pltpu.get_barrier_semaphore()
        for nbr in (left, right):
            pl.semaphore_signal(barrier, inc=1, device_id=(nbr,),
                                device_id_type=pl.DeviceIdType.MESH)
        pl.semaphore_wait(barrier, 2)

        def partial_half(c, half):
            xc = x_ref[pl.ds(c * CH, CH), :]
            wc = w_ref[:, pl.ds(half * N2, N2)]
            return jnp.dot(xc, wc, preferred_element_type=jnp.float32
                           ).astype(jnp.bfloat16)

        rings = [
            dict(comm=comm_a, stage=stage_a, send=send_a, recv=recv_a,
                 credit=credit_a, osem=osem_a, dst=right, ups=left,
                 sgn=1, half=0, pend=[None, None], n_store=[0], rdmas=[]),
            dict(comm=comm_b, stage=stage_b, send=send_b, recv=recv_b,
                 credit=credit_b, osem=osem_b, dst=left, ups=right,
                 sgn=-1, half=1, pend=[None, None], n_store=[0], rdmas=[]),
        ]

        for r in rings:
            r["comm"][0, :, :] = partial_half(me, r["half"])

        def relu_store(r, slot, c):
            st = r["n_store"][0] % 2
            if r["pend"][st] is not None:
                r["pend"][st].wait()
            r["stage"][st, :, :] = jnp.maximum(
                r["comm"][slot, :, :].astype(jnp.float32), 0.0)
            cp = pltpu.make_async_copy(
                r["stage"].at[st],
                out_ref.at[pl.ds(c * CH, CH), pl.ds(r["half"] * N2, N2)],
                r["osem"].at[st])
            cp.start()
            r["pend"][st] = cp
            r["n_store"][0] += 1

        for u in range(N_STEPS):
            s_slot = u % NSLOT
            r_slot = (u + 1) % NSLOT
            for r in rings:
                if u >= NSLOT - 1:
                    pl.semaphore_wait(r["credit"], 1)
                rdma = pltpu.make_async_remote_copy(
                    src_ref=r["comm"].at[s_slot],
                    dst_ref=r["comm"].at[r_slot],
                    send_sem=r["send"].at[s_slot],
                    recv_sem=r["recv"].at[r_slot],
                    device_id=(r["dst"],),
                    device_id_type=pl.DeviceIdType.MESH,
                )
                rdma.start()
                r["rdmas"].append(rdma)

            padd = []
            if u < N_DEV - 1:
                for r in rings:
                    c = (me - r["sgn"] * (u + 1)) % N_DEV
                    padd.append(partial_half(c, r["half"]))
            elif u == N_DEV - 1:
                for r in rings:
                    relu_store(r, s_slot, (me + r["sgn"]) % N_DEV)

            if u >= 1:
                for r in rings:
                    r["rdmas"][u - 1].wait_send()
                    if u - 1 <= N_STEPS - NSLOT:
                        pl.semaphore_signal(r["credit"], inc=1,
                                            device_id=(r["ups"],),
                                            device_id_type=pl.DeviceIdType.MESH)

            for r in rings:
                r["rdmas"][u].wait_recv()

            if u < N_DEV - 1:
                for r, p in zip(rings, padd):
                    r["comm"][r_slot, :, :] = r["comm"][r_slot, :, :] + p
            else:
                t = u - (N_DEV - 1)
                for r in rings:
                    c = (me - r["sgn"] * t) % N_DEV
                    relu_store(r, r_slot, c)

        for r in rings:
            r["rdmas"][N_STEPS - 1].wait_send()

        for r in rings:
            for st in range(2):
                if r["pend"][st] is not None:
                    r["pend"][st].wait()

    return pl.pallas_call(
        body,
        out_shape=jax.ShapeDtypeStruct((M, N), jnp.float32),
        in_specs=[
            pl.BlockSpec(memory_space=pltpu.VMEM),
            pl.BlockSpec(memory_space=pltpu.VMEM),
        ],
        out_specs=pl.BlockSpec(memory_space=pl.ANY),
        scratch_shapes=[
            pltpu.VMEM((NSLOT, CH, N2), jnp.bfloat16),
            pltpu.VMEM((NSLOT, CH, N2), jnp.bfloat16),
            pltpu.VMEM((2, CH, N2), jnp.float32),
            pltpu.VMEM((2, CH, N2), jnp.float32),
            pltpu.SemaphoreType.DMA((NSLOT,)),
            pltpu.SemaphoreType.DMA((NSLOT,)),
            pltpu.SemaphoreType.DMA((NSLOT,)),
            pltpu.SemaphoreType.DMA((NSLOT,)),
            pltpu.SemaphoreType.REGULAR,
            pltpu.SemaphoreType.REGULAR,
            pltpu.SemaphoreType.DMA((2,)),
            pltpu.SemaphoreType.DMA((2,)),
        ],
        compiler_params=pltpu.CompilerParams(
            collective_id=0, vmem_limit_bytes=100 * 1024 * 1024),
    )(x, w_mat)


# device time: 777820 ns/iter; 1.1395x vs baseline; 1.1395x over previous
import jax
import jax.numpy as jnp
from jax import lax
from jax.experimental import pallas as pl
from jax.experimental.pallas import tpu as pltpu

N_DEV = 16
NRING = 4
NSLOT = 4


def kernel(x, w_mat):
    M, K = x.shape
    _, N = w_mat.shape
    CH = M // N_DEV
    NQ = N // NRING
    N_STEPS = 2 * (N_DEV - 1)

    def body(x_ref, w_ref, out_ref, *scratch):
        comms = scratch[0:4]
        stages = scratch[4:8]
        sends = scratch[8:12]
        recvs = scratch[12:16]
        credits = scratch[16:20]
        osems = scratch[20:24]

        me = lax.axis_index("i")
        left = (me - 1) % N_DEV
        right = (me + 1) % N_DEV

        barrier = pltpu.get_barrier_semaphore()
        for nbr in (left, right):
            pl.semaphore_signal(barrier, inc=1, device_id=(nbr,),
                                device_id_type=pl.DeviceIdType.MESH)
        pl.semaphore_wait(barrier, 2)

        rings = []
        for q in range(NRING):
            sgn = 1 if q < NRING // 2 else -1
            rings.append(dict(
                q=q, sgn=sgn,
                dst=right if sgn == 1 else left,
                ups=left if sgn == 1 else right,
                comm=comms[q], stage=stages[q], send=sends[q],
                recv=recvs[q], credit=credits[q], osem=osems[q],
                rdmas=[], padd={}, pend=[None, None], n_store=[0]))
        order = [rings[0], rings[2], rings[1], rings[3]]

        def partial_q(c, q):
            xc = x_ref[pl.ds(c * CH, CH), :]
            wc = w_ref[:, pl.ds(q * NQ, NQ)]
            return jnp.dot(xc, wc, preferred_element_type=jnp.float32
                           ).astype(jnp.bfloat16)

        for r in rings:
            r["comm"][0, :, :] = partial_q(me, r["q"])

        def relu_store(r, slot, c):
            st = r["n_store"][0] % 2
            if r["pend"][st] is not None:
                r["pend"][st].wait()
            r["stage"][st, :, :] = jnp.maximum(
                r["comm"][slot, :, :].astype(jnp.float32), 0.0)
            cp = pltpu.make_async_copy(
                r["stage"].at[st],
                out_ref.at[pl.ds(c * CH, CH), pl.ds(r["q"] * NQ, NQ)],
                r["osem"].at[st])
            cp.start()
            r["pend"][st] = cp
            r["n_store"][0] += 1

        def launch(r, u):
            if u >= NSLOT - 1:
                pl.semaphore_wait(r["credit"], 1)
            s_slot = u % NSLOT
            r_slot = (u + 1) % NSLOT
            rdma = pltpu.make_async_remote_copy(
                src_ref=r["comm"].at[s_slot],
                dst_ref=r["comm"].at[r_slot],
                send_sem=r["send"].at[s_slot],
                recv_sem=r["recv"].at[r_slot],
                device_id=(r["dst"],),
                device_id_type=pl.DeviceIdType.MESH,
            )
            rdma.start()
            r["rdmas"].append(rdma)
            if u < N_DEV - 1:
                c = (me - r["sgn"] * (u + 1)) % N_DEV
                r["padd"][u] = partial_q(c, r["q"])
            elif u == N_DEV - 1:
                relu_store(r, s_slot, (me + r["sgn"]) % N_DEV)
            if u >= 1:
                r["rdmas"][u - 1].wait_send()
                if u - 1 <= N_STEPS - NSLOT:
                    pl.semaphore_signal(r["credit"], inc=1,
                                        device_id=(r["ups"],),
                                        device_id_type=pl.DeviceIdType.MESH)

        def finish(r, v):
            r["rdmas"][v].wait_recv()
            r_slot = (v + 1) % NSLOT
            if v < N_DEV - 1:
                r["comm"][r_slot, :, :] = r["comm"][r_slot, :, :] + r["padd"][v]
            else:
                t = v - (N_DEV - 1)
                c = (me - r["sgn"] * t) % N_DEV
                relu_store(r, r_slot, c)

        for u in range(N_STEPS + 1):
            for r in order:
                if u >= 1:
                    finish(r, u - 1)
                if u < N_STEPS:
                    launch(r, u)

        for r in rings:
            r["rdmas"][N_STEPS - 1].wait_send()
            for st in range(2):
                if r["pend"][st] is not None:
                    r["pend"][st].wait()

    return pl.pallas_call(
        body,
        out_shape=jax.ShapeDtypeStruct((M, N), jnp.float32),
        in_specs=[
            pl.BlockSpec(memory_space=pltpu.VMEM),
            pl.BlockSpec(memory_space=pltpu.VMEM),
        ],
        out_specs=pl.BlockSpec(memory_space=pl.ANY),
        scratch_shapes=(
            [pltpu.VMEM((NSLOT, CH, NQ), jnp.bfloat16)] * NRING +
            [pltpu.VMEM((2, CH, NQ), jnp.float32)] * NRING +
            [pltpu.SemaphoreType.DMA((NSLOT,))] * NRING +
            [pltpu.SemaphoreType.DMA((NSLOT,))] * NRING +
            [pltpu.SemaphoreType.REGULAR] * NRING +
            [pltpu.SemaphoreType.DMA((2,))] * NRING
        ),
        compiler_params=pltpu.CompilerParams(
            collective_id=0, vmem_limit_bytes=100 * 1024 * 1024),
    )(x, w_mat)
